# Initial kernel scaffold; baseline (speedup 1.0000x reference)
import jax
import jax.numpy as jnp
from jax.experimental import pallas as pl


def kernel(x, t, emb_table, W, b):
    raise NotImplementedError("write your pallas kernel here")


# speedup vs baseline: 1.9027x; 1.9027x over previous
import functools

import jax
import jax.numpy as jnp
from jax import lax
from jax.experimental import pallas as pl
from jax.experimental.pallas import tpu as pltpu
from jax.experimental.pallas import tpu_sc as plsc

N_HID = 128
MAX_LEN = 240
LANES = 16
CHUNK = 128


def _proj_table_body(emb_ref, w_ref, b_ref, out_ref):
    p = lax.dot_general(
        emb_ref[...], w_ref[...],
        dimension_numbers=(((1,), (1,)), ((), ())),
        preferred_element_type=jnp.float32,
    )
    out_ref[...] = p + b_ref[...]


@functools.cache
def _make_sc_kernel(n_rows):
    n_chunks = n_rows // CHUNK
    info = plsc.get_sparse_core_info()
    nc, ns = info.num_cores, info.num_subcores
    nw = nc * ns
    per_w = (n_chunks + nw - 1) // nw

    mesh = plsc.VectorSubcoreMesh(core_axis_name="c", subcore_axis_name="s")

    @functools.partial(
        pl.kernel,
        mesh=mesh,
        out_type=jax.ShapeDtypeStruct((n_rows, N_HID), jnp.float32),
        scratch_types=[
            pltpu.VMEM((CHUNK,), jnp.int32),
            pltpu.VMEM((CHUNK, N_HID), jnp.float32),
            pltpu.VMEM((CHUNK, N_HID), jnp.float32),
            pltpu.SemaphoreType.DMA,
            pltpu.SemaphoreType.DMA,
        ],
    )
    def sc_fn(x_hbm, t_hbm, p_hbm, out_hbm, iv, xv, ev, sem_x, sem_g):
        wid = lax.axis_index("s") * nc + lax.axis_index("c")

        def chunk_body(k, carry):
            cid = wid + k * nw

            @pl.when(cid < n_chunks)
            def _():
                base = cid * CHUNK
                pltpu.sync_copy(t_hbm.at[pl.ds(base, CHUNK)], iv)
                cx = pltpu.async_copy(x_hbm.at[pl.ds(base, CHUNK)], xv, sem_x)
                cg = pltpu.async_copy(p_hbm.at[iv], ev, sem_g)
                cx.wait()
                cg.wait()

                def row_body(i, c2):
                    for j in range(N_HID // LANES):
                        sl = pl.ds(j * LANES, LANES)
                        plsc.addupdate(xv.at[i, sl], ev[i, sl])
                    return c2

                lax.fori_loop(0, CHUNK, row_body, 0)
                pltpu.sync_copy(xv, out_hbm.at[pl.ds(base, CHUNK)])

            return carry

        lax.fori_loop(0, per_w, chunk_body, 0)

    return sc_fn


def kernel(x, t, emb_table, W, b):
    p_table = pl.pallas_call(
        _proj_table_body,
        out_shape=jax.ShapeDtypeStruct((MAX_LEN, N_HID), jnp.float32),
    )(emb_table, W, b.reshape(1, N_HID))
    return _make_sc_kernel(x.shape[0])(x, t, p_table)


# speedup vs baseline: 5.5417x; 2.9126x over previous
import functools

import jax
import jax.numpy as jnp
from jax import lax
from jax.experimental import pallas as pl
from jax.experimental.pallas import tpu as pltpu
from jax.experimental.pallas import tpu_sc as plsc

N_HID = 128
MAX_LEN = 240
LANES = 16
CHUNK = 128
MAX_PW = 79


def _proj_table_body(emb_ref, w_ref, b_ref, out_ref):
    p = lax.dot_general(
        emb_ref[...], w_ref[...],
        dimension_numbers=(((1,), (1,)), ((), ())),
        preferred_element_type=jnp.float32,
    )
    out_ref[...] = p + b_ref[...]


@functools.cache
def _make_sc_kernel(n_rows):
    n_chunks = n_rows // CHUNK
    info = plsc.get_sparse_core_info()
    nc, ns = info.num_cores, info.num_subcores
    nw = nc * ns

    mesh = plsc.VectorSubcoreMesh(core_axis_name="c", subcore_axis_name="s")

    @functools.partial(
        pl.kernel,
        mesh=mesh,
        out_type=jax.ShapeDtypeStruct((n_rows, N_HID), jnp.float32),
        scratch_types=[
            pltpu.VMEM((CHUNK,), jnp.int32),
            pltpu.VMEM((CHUNK,), jnp.int32),
            pltpu.VMEM((CHUNK,), jnp.int32),
            pltpu.VMEM((CHUNK,), jnp.int32),
            pltpu.VMEM((CHUNK, N_HID), jnp.float32),
            pltpu.VMEM((CHUNK, N_HID), jnp.float32),
            pltpu.VMEM((CHUNK, N_HID), jnp.float32),
            pltpu.VMEM((CHUNK, N_HID), jnp.float32),
            pltpu.VMEM((CHUNK, N_HID), jnp.float32),
            pltpu.VMEM((CHUNK, N_HID), jnp.float32),
            pltpu.VMEM_SHARED((MAX_LEN, N_HID), jnp.float32),
            pltpu.SemaphoreType.DMA,
            pltpu.SemaphoreType.DMA,
            pltpu.SemaphoreType.DMA,
            pltpu.SemaphoreType.DMA,
            pltpu.SemaphoreType.DMA,
            pltpu.SemaphoreType.DMA,
            pltpu.SemaphoreType.DMA,
            pltpu.SemaphoreType.DMA,
            pltpu.SemaphoreType.DMA,
            pltpu.SemaphoreType.DMA,
        ],
    )
    def sc_fn(x_hbm, t_hbm, p_hbm, out_hbm,
              iv0, iv1, iv2, iv3, xv0, xv1, ev0, ev1, ov0, ov1, p_sh,
              si0, si1, si2, si3, sx0, sx1, sg0, sg1, so0, so1):
        wid = lax.axis_index("s") * nc + lax.axis_index("c")

        iv = (iv0, iv1, iv2, iv3)
        si = (si0, si1, si2, si3)
        xv = (xv0, xv1)
        ev = (ev0, ev1)
        ov = (ov0, ov1)
        sx = (sx0, sx1)
        sg = (sg0, sg1)
        so = (so0, so1)

        @pl.when(lax.axis_index("s") == 0)
        def _():
            pltpu.sync_copy(p_hbm, p_sh)
        plsc.subcore_barrier()

        def valid(m):
            return wid + m * nw < n_chunks

        def row_base(m):
            return (wid + m * nw) * CHUNK

        def issue_idx(m, s4):
            @pl.when(valid(m))
            def _():
                pltpu.async_copy(
                    t_hbm.at[pl.ds(row_base(m), CHUNK)], iv[s4], si[s4])

        def issue_xg(m, s4, s2):
            @pl.when(valid(m))
            def _():
                pltpu.make_async_copy(
                    t_hbm.at[pl.ds(row_base(m), CHUNK)], iv[s4], si[s4]).wait()
                pltpu.async_copy(
                    x_hbm.at[pl.ds(row_base(m), CHUNK)], xv[s2], sx[s2])
                pltpu.async_copy(p_sh.at[iv[s4]], ev[s2], sg[s2])

        def crunch(m, s4, s2):
            @pl.when(valid(m))
            def _():
                rb = row_base(m)
                pltpu.make_async_copy(
                    x_hbm.at[pl.ds(rb, CHUNK)], xv[s2], sx[s2]).wait()
                pltpu.make_async_copy(p_sh.at[iv[s4]], ev[s2], sg[s2]).wait()

                @pl.when(m >= 2)
                def _():
                    pltpu.make_async_copy(
                        ov[s2], out_hbm.at[pl.ds(rb, CHUNK)], so[s2]).wait()

                def row_body(i, c):
                    for j in range(N_HID // LANES):
                        sl = pl.ds(j * LANES, LANES)
                        ov[s2][i, sl] = xv[s2][i, sl] + ev[s2][i, sl]
                    return c

                lax.fori_loop(0, CHUNK, row_body, 0)
                pltpu.async_copy(ov[s2], out_hbm.at[pl.ds(rb, CHUNK)], so[s2])

        issue_idx(0, 0)
        issue_idx(1, 1)
        issue_idx(2, 2)
        issue_idx(3, 3)
        issue_xg(0, 0, 0)

        def body4(g, carry):
            for dm in range(4):
                m = g * 4 + dm
                issue_xg(m + 1, (dm + 1) % 4, (dm + 1) % 2)
                crunch(m, dm, dm % 2)
                issue_idx(m + 4, dm)
            return carry

        lax.fori_loop(0, (MAX_PW + 3) // 4, body4, 0)

        pltpu.make_async_copy(ov[0], out_hbm.at[pl.ds(0, CHUNK)], so[0]).wait()
        pltpu.make_async_copy(ov[1], out_hbm.at[pl.ds(0, CHUNK)], so[1]).wait()

    return sc_fn


def kernel(x, t, emb_table, W, b):
    p_table = pl.pallas_call(
        _proj_table_body,
        out_shape=jax.ShapeDtypeStruct((MAX_LEN, N_HID), jnp.float32),
    )(emb_table, W, b.reshape(1, N_HID))
    return _make_sc_kernel(x.shape[0])(x, t, p_table)
